# baseline (speedup 1.0000x reference)
import jax
import jax.numpy as jnp
from jax import lax
from jax.experimental import pallas as pl
from jax.experimental.pallas import tpu as pltpu
from jax.experimental.pallas import tpu_sc as plsc

FACTORS = 16
BATCH = 16384
LANES = 16

_INFO = plsc.get_sparse_core_info()
NUM_CORES = _INFO.num_cores
NUM_SUBCORES = _INFO.num_subcores
NUM_WORKERS = NUM_CORES * NUM_SUBCORES

ROWS_PER_WORKER = BATCH // NUM_WORKERS
CHUNK = 128
NUM_CHUNKS = ROWS_PER_WORKER // CHUNK
GROUPS = ROWS_PER_WORKER // LANES


def _gmf_body(user_table, item_table, w_hbm, b_hbm, uidx_hbm, iidx_hbm,
              out_hbm, uidx, iidx, urows, irows, wv, bv, outv, sem):
    wid = lax.axis_index("s") * NUM_CORES + lax.axis_index("c")
    base = wid * ROWS_PER_WORKER

    pltpu.sync_copy(uidx_hbm.at[wid], uidx)
    pltpu.sync_copy(iidx_hbm.at[wid], iidx)
    pltpu.sync_copy(w_hbm, wv)
    pltpu.sync_copy(b_hbm, bv)

    copies = []
    for c in range(NUM_CHUNKS):
        copies.append(pltpu.async_copy(
            user_table.at[uidx.at[c]], urows.at[pl.ds(c * CHUNK, CHUNK)], sem))
        copies.append(pltpu.async_copy(
            item_table.at[iidx.at[c]], irows.at[pl.ds(c * CHUNK, CHUNK)], sem))
    for cp in copies:
        cp.wait()

    bvec = bv[...]
    wsplat = [wv[f, :] for f in range(FACTORS)]

    def group(g, carry):
        rows = g * LANES + lax.iota(jnp.int32, LANES)
        acc = bvec
        for f in range(FACTORS):
            col = jnp.full((LANES,), f, jnp.int32)
            uf = plsc.load_gather(urows, [rows, col])
            vf = plsc.load_gather(irows, [rows, col])
            acc = acc + uf * vf * wsplat[f]
        outv[pl.ds(g * LANES, LANES)] = 1.0 / (1.0 + jnp.exp(-acc))
        return carry

    lax.fori_loop(0, GROUPS, group, 0)
    pltpu.sync_copy(outv, out_hbm.at[pl.ds(base, ROWS_PER_WORKER)])


def kernel(user_table, item_table, W, b, user_indices, item_indices):
    w_splat = jnp.broadcast_to(W.reshape(FACTORS, 1), (FACTORS, LANES)).astype(jnp.float32)
    b_vec = jnp.broadcast_to(b.reshape(()), (LANES,)).astype(jnp.float32)
    uidx = user_indices.astype(jnp.int32).reshape(NUM_WORKERS, NUM_CHUNKS, CHUNK)
    iidx = item_indices.astype(jnp.int32).reshape(NUM_WORKERS, NUM_CHUNKS, CHUNK)

    run = pl.kernel(
        _gmf_body,
        mesh=plsc.VectorSubcoreMesh(core_axis_name="c", subcore_axis_name="s"),
        compiler_params=pltpu.CompilerParams(
            needs_layout_passes=False, use_tc_tiling_on_sc=False),
        out_type=jax.ShapeDtypeStruct((BATCH,), jnp.float32),
        scratch_types=[
            pltpu.VMEM((NUM_CHUNKS, CHUNK), jnp.int32),
            pltpu.VMEM((NUM_CHUNKS, CHUNK), jnp.int32),
            pltpu.VMEM((ROWS_PER_WORKER, FACTORS), jnp.float32),
            pltpu.VMEM((ROWS_PER_WORKER, FACTORS), jnp.float32),
            pltpu.VMEM((FACTORS, LANES), jnp.float32),
            pltpu.VMEM((LANES,), jnp.float32),
            pltpu.VMEM((ROWS_PER_WORKER,), jnp.float32),
            pltpu.SemaphoreType.DMA,
        ],
    )
    out = run(user_table, item_table, w_splat, b_vec, uidx, iidx)
    return out.reshape(BATCH, 1)
